# baseline (speedup 1.0000x reference)
import jax
import jax.numpy as jnp
from jax.experimental import pallas as pl
from jax.experimental.pallas import tpu as pltpu

N = 64
S = 16
C = 2048
BB = 32


def _select_one(counts, counts_w):
    present = counts > 0
    present_w = counts_w > 0
    U = jnp.sum(present.astype(jnp.int32))

    v_i = jax.lax.broadcasted_iota(jnp.int32, (N, N), 0)
    w_i = jax.lax.broadcasted_iota(jnp.int32, (N, N), 1)
    rank_w = jnp.sum(
        jnp.where(present & (v_i < w_i), 1, 0), axis=0, keepdims=True
    )

    lane_id = jax.lax.broadcasted_iota(jnp.int32, (1, N), 1)
    sub_id = jax.lax.broadcasted_iota(jnp.int32, (N, 1), 0)
    key_w = counts_w * N + (N - 1 - lane_id)
    key_v = counts * N + (N - 1 - sub_id)
    R_w = jnp.sum(jnp.where(key_v > key_w, 1, 0), axis=0, keepdims=True)

    j_i = jax.lax.broadcasted_iota(jnp.int32, (S, N), 0)
    w2_i = jax.lax.broadcasted_iota(jnp.int32, (S, N), 1)
    pad = S - U
    rtarget = jnp.maximum(j_i - pad, 0)
    m_pad = present_w & (rank_w == rtarget)
    out_pad = jnp.sum(jnp.where(m_pad, w2_i, 0), axis=1, keepdims=True)
    m_top = R_w == j_i
    out_top = jnp.sum(jnp.where(m_top, w2_i, 0), axis=1, keepdims=True)
    idx = jnp.where(U < S, out_pad, out_top)
    return (idx == w2_i).astype(jnp.float32)


def _kern(f_ref, v_ref, fout_ref, vout_ref):
    iota_vc = jax.lax.broadcasted_iota(jnp.int32, (N, C), 0)
    ones_c = jnp.ones((C, 1), jnp.float32)

    counts_cols = []
    for i in range(BB):
        F = f_ref[i]
        M = jnp.max(F, axis=0, keepdims=True)
        amax = jnp.min(jnp.where(F == M, iota_vc, N), axis=0, keepdims=True)
        eq = (amax == iota_vc).astype(jnp.float32)
        counts_cols.append(
            jnp.dot(eq, ones_c, preferred_element_type=jnp.float32))

    counts_mat = jnp.concatenate(counts_cols, axis=1).astype(jnp.int32)
    counts_mat_t = jnp.transpose(counts_mat)

    for i in range(BB):
        onehot = _select_one(counts_mat[:, i:i + 1], counts_mat_t[i:i + 1, :])
        fout_ref[i] = jnp.dot(onehot, f_ref[i],
                              preferred_element_type=jnp.float32)
        vout_ref[i] = jnp.dot(onehot, v_ref[i],
                              preferred_element_type=jnp.float32)


def kernel(F0, vertices0, k):
    B = F0.shape[0]
    out_shapes = (
        jax.ShapeDtypeStruct((B, S, C), jnp.float32),
        jax.ShapeDtypeStruct((B, S, 3), jnp.float32),
    )
    F_new, vertices_new = pl.pallas_call(
        _kern,
        grid=(B // BB,),
        in_specs=[
            pl.BlockSpec((BB, N, C), lambda b: (b, 0, 0)),
            pl.BlockSpec((BB, N, 3), lambda b: (b, 0, 0)),
        ],
        out_specs=(
            pl.BlockSpec((BB, S, C), lambda b: (b, 0, 0)),
            pl.BlockSpec((BB, S, 3), lambda b: (b, 0, 0)),
        ),
        out_shape=out_shapes,
        compiler_params=pltpu.CompilerParams(
            dimension_semantics=("parallel",),
        ),
    )(F0, vertices0)
    return (F_new, vertices_new)


# speedup vs baseline: 1.0246x; 1.0246x over previous
import jax
import jax.numpy as jnp
from jax.experimental import pallas as pl
from jax.experimental.pallas import tpu as pltpu

N = 64
S = 16
C = 2048
BB = 16


def _select_one(counts, counts_w):
    present = counts > 0
    present_w = counts_w > 0
    U = jnp.sum(present.astype(jnp.int32))

    v_i = jax.lax.broadcasted_iota(jnp.int32, (N, N), 0)
    w_i = jax.lax.broadcasted_iota(jnp.int32, (N, N), 1)
    rank_w = jnp.sum(
        jnp.where(present & (v_i < w_i), 1, 0), axis=0, keepdims=True
    )

    lane_id = jax.lax.broadcasted_iota(jnp.int32, (1, N), 1)
    sub_id = jax.lax.broadcasted_iota(jnp.int32, (N, 1), 0)
    key_w = counts_w * N + (N - 1 - lane_id)
    key_v = counts * N + (N - 1 - sub_id)
    R_w = jnp.sum(jnp.where(key_v > key_w, 1, 0), axis=0, keepdims=True)

    j_i = jax.lax.broadcasted_iota(jnp.int32, (S, N), 0)
    w2_i = jax.lax.broadcasted_iota(jnp.int32, (S, N), 1)
    pad = S - U
    rtarget = jnp.maximum(j_i - pad, 0)
    m_pad = present_w & (rank_w == rtarget)
    out_pad = jnp.sum(jnp.where(m_pad, w2_i, 0), axis=1, keepdims=True)
    m_top = R_w == j_i
    out_top = jnp.sum(jnp.where(m_top, w2_i, 0), axis=1, keepdims=True)
    idx = jnp.where(U < S, out_pad, out_top)
    return (idx == w2_i).astype(jnp.float32)


def _kern(f_ref, v_ref, fout_ref, vout_ref):
    iota_vc = jax.lax.broadcasted_iota(jnp.int32, (N, C), 0)
    ones_c = jnp.ones((C, 1), jnp.float32)

    counts_cols = []
    for i in range(BB):
        F = f_ref[i]
        M = jnp.max(F, axis=0, keepdims=True)
        amax = jnp.min(jnp.where(F == M, iota_vc, N), axis=0, keepdims=True)
        eq = (amax == iota_vc).astype(jnp.float32)
        counts_cols.append(
            jnp.dot(eq, ones_c, preferred_element_type=jnp.float32))

    counts_mat = jnp.concatenate(counts_cols, axis=1).astype(jnp.int32)
    counts_mat_t = jnp.transpose(counts_mat)

    for i in range(BB):
        onehot = _select_one(counts_mat[:, i:i + 1], counts_mat_t[i:i + 1, :])
        fout_ref[i] = jnp.dot(onehot, f_ref[i],
                              preferred_element_type=jnp.float32)
        vout_ref[i] = jnp.dot(onehot, v_ref[i],
                              preferred_element_type=jnp.float32)


def kernel(F0, vertices0, k):
    B = F0.shape[0]
    out_shapes = (
        jax.ShapeDtypeStruct((B, S, C), jnp.float32),
        jax.ShapeDtypeStruct((B, S, 3), jnp.float32),
    )
    F_new, vertices_new = pl.pallas_call(
        _kern,
        grid=(B // BB,),
        in_specs=[
            pl.BlockSpec((BB, N, C), lambda b: (b, 0, 0)),
            pl.BlockSpec((BB, N, 3), lambda b: (b, 0, 0)),
        ],
        out_specs=(
            pl.BlockSpec((BB, S, C), lambda b: (b, 0, 0)),
            pl.BlockSpec((BB, S, 3), lambda b: (b, 0, 0)),
        ),
        out_shape=out_shapes,
        compiler_params=pltpu.CompilerParams(
            dimension_semantics=("parallel",),
        ),
    )(F0, vertices0)
    return (F_new, vertices_new)


# speedup vs baseline: 1.0644x; 1.0389x over previous
import jax
import jax.numpy as jnp
from jax.experimental import pallas as pl
from jax.experimental.pallas import tpu as pltpu

N = 64
S = 16
C = 2048
BB = 16


def _select_one(counts, counts_w):
    present = counts > 0
    present_w = counts_w > 0
    U = jnp.sum(present.astype(jnp.int32))

    v_i = jax.lax.broadcasted_iota(jnp.int32, (N, N), 0)
    w_i = jax.lax.broadcasted_iota(jnp.int32, (N, N), 1)
    rank_w = jnp.sum(
        jnp.where(present & (v_i < w_i), 1, 0), axis=0, keepdims=True
    )

    lane_id = jax.lax.broadcasted_iota(jnp.int32, (1, N), 1)
    sub_id = jax.lax.broadcasted_iota(jnp.int32, (N, 1), 0)
    key_w = counts_w * N + (N - 1 - lane_id)
    key_v = counts * N + (N - 1 - sub_id)
    R_w = jnp.sum(jnp.where(key_v > key_w, 1, 0), axis=0, keepdims=True)

    j_i = jax.lax.broadcasted_iota(jnp.int32, (S, N), 0)
    w2_i = jax.lax.broadcasted_iota(jnp.int32, (S, N), 1)
    pad = S - U
    rtarget = jnp.maximum(j_i - pad, 0)
    m_pad = present_w & (rank_w == rtarget)
    out_pad = jnp.sum(jnp.where(m_pad, w2_i, 0), axis=1, keepdims=True)
    m_top = R_w == j_i
    out_top = jnp.sum(jnp.where(m_top, w2_i, 0), axis=1, keepdims=True)
    idx = jnp.where(U < S, out_pad, out_top)
    return (idx == w2_i).astype(jnp.float32)


def _kern(f_ref, v_ref, fout_ref, vout_ref):
    iota_f = jax.lax.broadcasted_iota(jnp.int32, (N, C), 0).astype(jnp.float32)
    ones_c = jnp.ones((C, 1), jnp.float32)

    counts_cols = []
    for i in range(BB):
        F = f_ref[i]
        M = jnp.max(F, axis=0, keepdims=True)
        amax = jnp.min(jnp.where(F == M, iota_f, float(N)),
                       axis=0, keepdims=True)
        eq = (amax == iota_f).astype(jnp.float32)
        counts_cols.append(
            jnp.dot(eq, ones_c, preferred_element_type=jnp.float32))

    counts_mat = jnp.concatenate(counts_cols, axis=1).astype(jnp.int32)
    counts_mat_t = jnp.transpose(counts_mat)

    for i in range(BB):
        onehot = _select_one(counts_mat[:, i:i + 1], counts_mat_t[i:i + 1, :])
        fout_ref[i] = jnp.dot(onehot, f_ref[i],
                              preferred_element_type=jnp.float32)
        vout_ref[i] = jnp.dot(onehot, v_ref[i],
                              preferred_element_type=jnp.float32)


def kernel(F0, vertices0, k):
    B = F0.shape[0]
    out_shapes = (
        jax.ShapeDtypeStruct((B, S, C), jnp.float32),
        jax.ShapeDtypeStruct((B, S, 3), jnp.float32),
    )
    F_new, vertices_new = pl.pallas_call(
        _kern,
        grid=(B // BB,),
        in_specs=[
            pl.BlockSpec((BB, N, C), lambda b: (b, 0, 0)),
            pl.BlockSpec((BB, N, 3), lambda b: (b, 0, 0)),
        ],
        out_specs=(
            pl.BlockSpec((BB, S, C), lambda b: (b, 0, 0)),
            pl.BlockSpec((BB, S, 3), lambda b: (b, 0, 0)),
        ),
        out_shape=out_shapes,
        compiler_params=pltpu.CompilerParams(
            dimension_semantics=("parallel",),
        ),
    )(F0, vertices0)
    return (F_new, vertices_new)
